# baseline (speedup 1.0000x reference)
import functools

import jax
import jax.numpy as jnp
from jax import lax
from jax.experimental import pallas as pl
from jax.experimental.pallas import tpu as pltpu
from jax.experimental.pallas import tpu_sc as plsc

_N = 10000
_E = 320000
_D = 128
_DH = 64

_NC = 2
_NS = 16
_CHUNK = 128
_BCH = 40
_NBANK = 4
_CPW = _NBANK * _BCH
_EPAD = _NS * _CPW * _CHUNK
_ACC_ROWS = 10256
_ZROWS = 648
_TAIL = _ACC_ROWS - 15 * _ZROWS
_XROWS = 632
_XTAIL = _N - 15 * _XROWS


@functools.partial(
    pl.kernel,
    mesh=plsc.VectorSubcoreMesh(core_axis_name="c", subcore_axis_name="s"),
    compiler_params=pltpu.CompilerParams(use_tc_tiling_on_sc=False),
    out_type=jax.ShapeDtypeStruct((_ACC_ROWS, _D), jnp.float32),
    scratch_types=[
        pltpu.VMEM((2, _BCH, _CHUNK), jnp.int32),
        pltpu.VMEM((2, _BCH, _CHUNK), jnp.int32),
        pltpu.VMEM((2, _CHUNK, _DH), jnp.float32),
        pltpu.VMEM_SHARED((_N, _DH), jnp.float32),
        pltpu.VMEM_SHARED((_ACC_ROWS, _DH), jnp.float32),
        pltpu.SemaphoreType.DMA,
        pltpu.SemaphoreType.DMA,
        pltpu.SemaphoreType.DMA,
        pltpu.SemaphoreType.DMA,
    ],
)
def _sc_agg(xc_hbm, src_hbm, dst_hbm, zeros_hbm, out_hbm,
            src_v, dst_v, rows_v, x_sh, acc_sh, sem0, sem1, bsem0, bsem1):
    sems = (sem0, sem1)
    bsems = (bsem0, bsem1)
    cid = lax.axis_index("c")
    sid = lax.axis_index("s")

    @pl.when(sid < 15)
    def _():
        pltpu.sync_copy(xc_hbm.at[cid, pl.ds(sid * _XROWS, _XROWS)],
                        x_sh.at[pl.ds(sid * _XROWS, _XROWS)])
        pltpu.sync_copy(zeros_hbm, acc_sh.at[pl.ds(sid * _ZROWS, _ZROWS)])

    @pl.when(sid == 15)
    def _():
        pltpu.sync_copy(xc_hbm.at[cid, pl.ds(15 * _XROWS, _XTAIL)],
                        x_sh.at[pl.ds(15 * _XROWS, _XTAIL)])
        pltpu.sync_copy(zeros_hbm.at[pl.ds(0, _TAIL)],
                        acc_sh.at[pl.ds(15 * _ZROWS, _TAIL)])

    for k in range(2):
        pltpu.async_copy(src_hbm.at[sid, k], src_v.at[k], bsems[k])
        pltpu.async_copy(dst_hbm.at[sid, k], dst_v.at[k], bsems[k])
    plsc.subcore_barrier()

    for k in range(_NBANK):
        s = k % 2
        pltpu.make_async_copy(src_hbm.at[sid, k], src_v.at[s], bsems[s]).wait()
        pltpu.make_async_copy(dst_hbm.at[sid, k], dst_v.at[s], bsems[s]).wait()

        for b in range(2):
            pltpu.async_copy(x_sh.at[src_v.at[s, b]], rows_v.at[b], sems[b])

        def step(g, carry):
            for b in range(2):
                j = g * 2 + b
                pltpu.make_async_copy(
                    x_sh.at[src_v.at[s, j]], rows_v.at[b], sems[b]).wait()
                pltpu.sync_copy(rows_v.at[b], acc_sh.at[dst_v.at[s, j]],
                                add=True)
                pltpu.async_copy(
                    x_sh.at[src_v.at[s, j + 2]], rows_v.at[b], sems[b])
            return carry

        lax.fori_loop(0, _BCH // 2 - 1, step, 0)
        for b in range(2):
            j = _BCH - 2 + b
            pltpu.make_async_copy(
                x_sh.at[src_v.at[s, j]], rows_v.at[b], sems[b]).wait()
            pltpu.sync_copy(rows_v.at[b], acc_sh.at[dst_v.at[s, j]], add=True)
        if k + 2 < _NBANK:
            pltpu.async_copy(src_hbm.at[sid, k + 2], src_v.at[s], bsems[s])
            pltpu.async_copy(dst_hbm.at[sid, k + 2], dst_v.at[s], bsems[s])

    plsc.subcore_barrier()

    @pl.when(sid < 15)
    def _():
        pltpu.sync_copy(acc_sh.at[pl.ds(sid * _ZROWS, _ZROWS)],
                        out_hbm.at[pl.ds(sid * _ZROWS, _ZROWS),
                                   pl.ds(cid * _DH, _DH)])

    @pl.when(sid == 15)
    def _():
        pltpu.sync_copy(acc_sh.at[pl.ds(15 * _ZROWS, _TAIL)],
                        out_hbm.at[pl.ds(15 * _ZROWS, _TAIL),
                                   pl.ds(cid * _DH, _DH)])


def _mlp_body(x_ref, p_ref, w1_ref, b1_ref, w2_ref, b2_ref, g_ref, be_ref,
              o_ref):
    h = x_ref[...] + p_ref[:_N]
    h = lax.dot_general(h, w1_ref[...], (((1,), (1,)), ((), ())),
                        preferred_element_type=jnp.float32) + b1_ref[...]
    h = jnp.maximum(h, 0.0)
    h = lax.dot_general(h, w2_ref[...], (((1,), (1,)), ((), ())),
                        preferred_element_type=jnp.float32) + b2_ref[...]
    h = jnp.maximum(h, 0.0)
    mean = jnp.mean(h, axis=0, keepdims=True)
    var = jnp.mean(jnp.square(h - mean), axis=0, keepdims=True)
    o_ref[...] = (h - mean) * lax.rsqrt(var + 1e-5) * g_ref[...] + be_ref[...]


def kernel(x, edge_index, W1, b1, W2, b2, gamma, beta):
    src = edge_index[0].astype(jnp.int32)
    dst = edge_index[1].astype(jnp.int32)
    pad = _EPAD - _E
    src_p = jnp.concatenate([src, jnp.zeros((pad,), jnp.int32)])
    dst_fill = _N + (jnp.arange(pad, dtype=jnp.int32) % (_ACC_ROWS - _N))
    dst_p = jnp.concatenate([dst, dst_fill])
    src_p = (src_p.reshape(_CPW, _NS, _CHUNK).transpose(1, 0, 2)
             .reshape(_NS, _NBANK, _BCH, _CHUNK))
    dst_p = (dst_p.reshape(_CPW, _NS, _CHUNK).transpose(1, 0, 2)
             .reshape(_NS, _NBANK, _BCH, _CHUNK))
    zeros = jnp.zeros((_ZROWS, _DH), jnp.float32)
    xc = x.reshape(_N, _NC, _DH).transpose(1, 0, 2)

    agg = _sc_agg(xc, src_p, dst_p, zeros)

    return pl.pallas_call(
        _mlp_body,
        out_shape=jax.ShapeDtypeStruct((_N, _D), jnp.float32),
    )(x, agg, W1, b1.reshape(1, _D), W2, b2.reshape(1, _D),
      gamma.reshape(1, _D), beta.reshape(1, _D))


# speedup vs baseline: 1.1550x; 1.1550x over previous
import functools

import jax
import jax.numpy as jnp
from jax import lax
from jax.experimental import pallas as pl
from jax.experimental.pallas import tpu as pltpu
from jax.experimental.pallas import tpu_sc as plsc

_N = 10000
_E = 320000
_D = 128
_DH = 64

_NC = 2
_NS = 16
_CHUNK = 128
_BCH = 40
_NBANK = 4
_CPW = _NBANK * _BCH
_EPAD = _NS * _CPW * _CHUNK
_ACC_ROWS = 10256
_ZROWS = 648
_TAIL = _ACC_ROWS - 15 * _ZROWS
_XROWS = 632
_XTAIL = _N - 15 * _XROWS


@functools.partial(
    pl.kernel,
    mesh=plsc.VectorSubcoreMesh(core_axis_name="c", subcore_axis_name="s"),
    compiler_params=pltpu.CompilerParams(use_tc_tiling_on_sc=False),
    out_type=jax.ShapeDtypeStruct((_ACC_ROWS, _D), jnp.float32),
    scratch_types=[
        pltpu.VMEM((2, _BCH, _CHUNK), jnp.int32),
        pltpu.VMEM((2, _BCH, _CHUNK), jnp.int32),
        pltpu.VMEM((3, _CHUNK, _DH), jnp.float32),
        pltpu.VMEM_SHARED((_N, _DH), jnp.float32),
        pltpu.VMEM_SHARED((_ACC_ROWS, _DH), jnp.float32),
        pltpu.SemaphoreType.DMA,
        pltpu.SemaphoreType.DMA,
        pltpu.SemaphoreType.DMA,
        pltpu.SemaphoreType.DMA,
        pltpu.SemaphoreType.DMA,
        pltpu.SemaphoreType.DMA,
        pltpu.SemaphoreType.DMA,
        pltpu.SemaphoreType.DMA,
    ],
)
def _sc_agg(xc_hbm, src_hbm, dst_hbm, zeros_hbm, out_hbm,
            src_v, dst_v, rows_v, x_sh, acc_sh,
            sem0, sem1, sem2, ssem0, ssem1, ssem2, bsem0, bsem1):
    sems = (sem0, sem1, sem2)
    ssems = (ssem0, ssem1, ssem2)
    bsems = (bsem0, bsem1)
    cid = lax.axis_index("c")
    sid = lax.axis_index("s")

    @pl.when(sid < 15)
    def _():
        pltpu.sync_copy(xc_hbm.at[cid, pl.ds(sid * _XROWS, _XROWS)],
                        x_sh.at[pl.ds(sid * _XROWS, _XROWS)])
        pltpu.sync_copy(zeros_hbm, acc_sh.at[pl.ds(sid * _ZROWS, _ZROWS)])

    @pl.when(sid == 15)
    def _():
        pltpu.sync_copy(xc_hbm.at[cid, pl.ds(15 * _XROWS, _XTAIL)],
                        x_sh.at[pl.ds(15 * _XROWS, _XTAIL)])
        pltpu.sync_copy(zeros_hbm.at[pl.ds(0, _TAIL)],
                        acc_sh.at[pl.ds(15 * _ZROWS, _TAIL)])

    for k in range(2):
        pltpu.async_copy(src_hbm.at[sid, k], src_v.at[k], bsems[k])
        pltpu.async_copy(dst_hbm.at[sid, k], dst_v.at[k], bsems[k])
    plsc.subcore_barrier()

    for k in range(_NBANK):
        s = k % 2
        pltpu.make_async_copy(src_hbm.at[sid, k], src_v.at[s], bsems[s]).wait()
        pltpu.make_async_copy(dst_hbm.at[sid, k], dst_v.at[s], bsems[s]).wait()

        for b in range(2):
            pltpu.async_copy(x_sh.at[src_v.at[s, b]], rows_v.at[b], sems[b])

        def step(g, carry):
            for b in range(3):
                j = g * 3 + b
                bn = (b + 2) % 3
                pltpu.make_async_copy(
                    x_sh.at[src_v.at[s, j]], rows_v.at[b], sems[b]).wait()
                pltpu.async_copy(rows_v.at[b], acc_sh.at[dst_v.at[s, j]],
                                 ssems[b], add=True)
                if b == 0:
                    @pl.when(g > 0)
                    def _():
                        pltpu.make_async_copy(
                            rows_v.at[bn], acc_sh.at[dst_v.at[s, j - 1]],
                            ssems[bn]).wait()
                else:
                    pltpu.make_async_copy(
                        rows_v.at[bn], acc_sh.at[dst_v.at[s, j - 1]],
                        ssems[bn]).wait()
                if b == 2:
                    @pl.when(g < _BCH // 3 - 1)
                    def _():
                        pltpu.async_copy(x_sh.at[src_v.at[s, j + 2]],
                                         rows_v.at[bn], sems[bn])
                else:
                    pltpu.async_copy(x_sh.at[src_v.at[s, j + 2]],
                                     rows_v.at[bn], sems[bn])
            return carry

        lax.fori_loop(0, _BCH // 3, step, 0)
        j = _BCH - 1
        pltpu.make_async_copy(
            x_sh.at[src_v.at[s, j]], rows_v.at[0], sems[0]).wait()
        pltpu.async_copy(rows_v.at[0], acc_sh.at[dst_v.at[s, j]],
                         ssems[0], add=True)
        pltpu.make_async_copy(
            rows_v.at[2], acc_sh.at[dst_v.at[s, j - 1]], ssems[2]).wait()
        pltpu.make_async_copy(
            rows_v.at[0], acc_sh.at[dst_v.at[s, j]], ssems[0]).wait()
        if k + 2 < _NBANK:
            pltpu.async_copy(src_hbm.at[sid, k + 2], src_v.at[s], bsems[s])
            pltpu.async_copy(dst_hbm.at[sid, k + 2], dst_v.at[s], bsems[s])

    plsc.subcore_barrier()

    @pl.when(sid < 15)
    def _():
        pltpu.sync_copy(acc_sh.at[pl.ds(sid * _ZROWS, _ZROWS)],
                        out_hbm.at[pl.ds(sid * _ZROWS, _ZROWS),
                                   pl.ds(cid * _DH, _DH)])

    @pl.when(sid == 15)
    def _():
        pltpu.sync_copy(acc_sh.at[pl.ds(15 * _ZROWS, _TAIL)],
                        out_hbm.at[pl.ds(15 * _ZROWS, _TAIL),
                                   pl.ds(cid * _DH, _DH)])


def _mlp_body(x_ref, p_ref, w1_ref, b1_ref, w2_ref, b2_ref, g_ref, be_ref,
              o_ref):
    h = x_ref[...] + p_ref[:_N]
    h = lax.dot_general(h, w1_ref[...], (((1,), (1,)), ((), ())),
                        preferred_element_type=jnp.float32) + b1_ref[...]
    h = jnp.maximum(h, 0.0)
    h = lax.dot_general(h, w2_ref[...], (((1,), (1,)), ((), ())),
                        preferred_element_type=jnp.float32) + b2_ref[...]
    h = jnp.maximum(h, 0.0)
    mean = jnp.mean(h, axis=0, keepdims=True)
    var = jnp.mean(jnp.square(h - mean), axis=0, keepdims=True)
    o_ref[...] = (h - mean) * lax.rsqrt(var + 1e-5) * g_ref[...] + be_ref[...]


def kernel(x, edge_index, W1, b1, W2, b2, gamma, beta):
    src = edge_index[0].astype(jnp.int32)
    dst = edge_index[1].astype(jnp.int32)
    pad = _EPAD - _E
    src_p = jnp.concatenate([src, jnp.zeros((pad,), jnp.int32)])
    dst_fill = _N + (jnp.arange(pad, dtype=jnp.int32) % (_ACC_ROWS - _N))
    dst_p = jnp.concatenate([dst, dst_fill])
    src_p = (src_p.reshape(_CPW, _NS, _CHUNK).transpose(1, 0, 2)
             .reshape(_NS, _NBANK, _BCH, _CHUNK))
    dst_p = (dst_p.reshape(_CPW, _NS, _CHUNK).transpose(1, 0, 2)
             .reshape(_NS, _NBANK, _BCH, _CHUNK))
    zeros = jnp.zeros((_ZROWS, _DH), jnp.float32)
    xc = x.reshape(_N, _NC, _DH).transpose(1, 0, 2)

    agg = _sc_agg(xc, src_p, dst_p, zeros)

    return pl.pallas_call(
        _mlp_body,
        out_shape=jax.ShapeDtypeStruct((_N, _D), jnp.float32),
    )(x, agg, W1, b1.reshape(1, _D), W2, b2.reshape(1, _D),
      gamma.reshape(1, _D), beta.reshape(1, _D))


# speedup vs baseline: 1.2794x; 1.1078x over previous
import functools

import jax
import jax.numpy as jnp
from jax import lax
from jax.experimental import pallas as pl
from jax.experimental.pallas import tpu as pltpu
from jax.experimental.pallas import tpu_sc as plsc

_N = 10000
_E = 320000
_D = 128
_DH = 64

_NC = 2
_NS = 16
_CHUNK = 128
_BCH = 40
_NBANK = 4
_CPW = _NBANK * _BCH
_EPAD = _NS * _CPW * _CHUNK
_ACC_ROWS = 10256
_ZROWS = 648
_TAIL = _ACC_ROWS - 15 * _ZROWS
_XROWS = 632
_XTAIL = _N - 15 * _XROWS


@functools.partial(
    pl.kernel,
    mesh=plsc.VectorSubcoreMesh(core_axis_name="c", subcore_axis_name="s"),
    compiler_params=pltpu.CompilerParams(use_tc_tiling_on_sc=False),
    out_type=jax.ShapeDtypeStruct((_ACC_ROWS, _D), jnp.float32),
    scratch_types=[
        pltpu.VMEM((2, _BCH, _CHUNK), jnp.int32),
        pltpu.VMEM((2, _BCH, _CHUNK), jnp.int32),
        pltpu.VMEM((3, _CHUNK, _DH), jnp.float32),
        pltpu.VMEM_SHARED((_N, _DH), jnp.float32),
        pltpu.VMEM_SHARED((_ACC_ROWS, _DH), jnp.float32),
        pltpu.SemaphoreType.DMA,
        pltpu.SemaphoreType.DMA,
        pltpu.SemaphoreType.DMA,
        pltpu.SemaphoreType.DMA,
        pltpu.SemaphoreType.DMA,
        pltpu.SemaphoreType.DMA,
        pltpu.SemaphoreType.DMA,
        pltpu.SemaphoreType.DMA,
    ],
)
def _sc_agg(xc_hbm, src_hbm, dst_hbm, zeros_hbm, out_hbm,
            src_v, dst_v, rows_v, x_sh, acc_sh,
            sem0, sem1, sem2, ssem0, ssem1, ssem2, bsem0, bsem1):
    sems = (sem0, sem1, sem2)
    ssems = (ssem0, ssem1, ssem2)
    bsems = (bsem0, bsem1)
    cid = lax.axis_index("c")
    sid = lax.axis_index("s")

    @pl.when(sid < 15)
    def _():
        pltpu.sync_copy(xc_hbm.at[pl.ds(sid * _XROWS, _XROWS),
                                  pl.ds(cid * _DH, _DH)],
                        x_sh.at[pl.ds(sid * _XROWS, _XROWS)])
        pltpu.sync_copy(zeros_hbm, acc_sh.at[pl.ds(sid * _ZROWS, _ZROWS)])

    @pl.when(sid == 15)
    def _():
        pltpu.sync_copy(xc_hbm.at[pl.ds(15 * _XROWS, _XTAIL),
                                  pl.ds(cid * _DH, _DH)],
                        x_sh.at[pl.ds(15 * _XROWS, _XTAIL)])
        pltpu.sync_copy(zeros_hbm.at[pl.ds(0, _TAIL)],
                        acc_sh.at[pl.ds(15 * _ZROWS, _TAIL)])

    for k in range(2):
        pltpu.async_copy(src_hbm.at[k, :, sid], src_v.at[k], bsems[k])
        pltpu.async_copy(dst_hbm.at[k, :, sid], dst_v.at[k], bsems[k])
    plsc.subcore_barrier()

    for k in range(_NBANK):
        s = k % 2
        pltpu.make_async_copy(src_hbm.at[k, :, sid], src_v.at[s],
                              bsems[s]).wait()
        pltpu.make_async_copy(dst_hbm.at[k, :, sid], dst_v.at[s],
                              bsems[s]).wait()

        for b in range(2):
            pltpu.async_copy(x_sh.at[src_v.at[s, b]], rows_v.at[b], sems[b])

        def step(g, carry):
            for b in range(3):
                j = g * 3 + b
                bn = (b + 2) % 3
                pltpu.make_async_copy(
                    x_sh.at[src_v.at[s, j]], rows_v.at[b], sems[b]).wait()
                pltpu.async_copy(rows_v.at[b], acc_sh.at[dst_v.at[s, j]],
                                 ssems[b], add=True)
                if b == 0:
                    @pl.when(g > 0)
                    def _():
                        pltpu.make_async_copy(
                            rows_v.at[bn], acc_sh.at[dst_v.at[s, j - 1]],
                            ssems[bn]).wait()
                else:
                    pltpu.make_async_copy(
                        rows_v.at[bn], acc_sh.at[dst_v.at[s, j - 1]],
                        ssems[bn]).wait()
                if b == 2:
                    @pl.when(g < _BCH // 3 - 1)
                    def _():
                        pltpu.async_copy(x_sh.at[src_v.at[s, j + 2]],
                                         rows_v.at[bn], sems[bn])
                else:
                    pltpu.async_copy(x_sh.at[src_v.at[s, j + 2]],
                                     rows_v.at[bn], sems[bn])
            return carry

        lax.fori_loop(0, _BCH // 3, step, 0)
        j = _BCH - 1
        pltpu.make_async_copy(
            x_sh.at[src_v.at[s, j]], rows_v.at[0], sems[0]).wait()
        pltpu.async_copy(rows_v.at[0], acc_sh.at[dst_v.at[s, j]],
                         ssems[0], add=True)
        pltpu.make_async_copy(
            rows_v.at[2], acc_sh.at[dst_v.at[s, j - 1]], ssems[2]).wait()
        pltpu.make_async_copy(
            rows_v.at[0], acc_sh.at[dst_v.at[s, j]], ssems[0]).wait()
        if k + 2 < _NBANK:
            pltpu.async_copy(src_hbm.at[k + 2, :, sid], src_v.at[s], bsems[s])
            pltpu.async_copy(dst_hbm.at[k + 2, :, sid], dst_v.at[s], bsems[s])

    plsc.subcore_barrier()

    @pl.when(sid < 15)
    def _():
        pltpu.sync_copy(acc_sh.at[pl.ds(sid * _ZROWS, _ZROWS)],
                        out_hbm.at[pl.ds(sid * _ZROWS, _ZROWS),
                                   pl.ds(cid * _DH, _DH)])

    @pl.when(sid == 15)
    def _():
        pltpu.sync_copy(acc_sh.at[pl.ds(15 * _ZROWS, _TAIL)],
                        out_hbm.at[pl.ds(15 * _ZROWS, _TAIL),
                                   pl.ds(cid * _DH, _DH)])


def _mlp_body(x_ref, p_ref, w1_ref, b1_ref, w2_ref, b2_ref, g_ref, be_ref,
              o_ref):
    h = x_ref[...] + p_ref[:_N]
    h = lax.dot_general(h, w1_ref[...], (((1,), (1,)), ((), ())),
                        preferred_element_type=jnp.float32) + b1_ref[...]
    h = jnp.maximum(h, 0.0)
    h = lax.dot_general(h, w2_ref[...], (((1,), (1,)), ((), ())),
                        preferred_element_type=jnp.float32) + b2_ref[...]
    h = jnp.maximum(h, 0.0)
    mean = jnp.mean(h, axis=0, keepdims=True)
    var = jnp.mean(jnp.square(h - mean), axis=0, keepdims=True)
    o_ref[...] = (h - mean) * lax.rsqrt(var + 1e-5) * g_ref[...] + be_ref[...]


def kernel(x, edge_index, W1, b1, W2, b2, gamma, beta):
    src = edge_index[0].astype(jnp.int32)
    dst = edge_index[1].astype(jnp.int32)
    pad = _EPAD - _E
    src_p = jnp.concatenate([src, jnp.zeros((pad,), jnp.int32)])
    dst_fill = _N + (jnp.arange(pad, dtype=jnp.int32) % (_ACC_ROWS - _N))
    dst_p = jnp.concatenate([dst, dst_fill])
    src_p = src_p.reshape(_NBANK, _BCH, _NS, _CHUNK)
    dst_p = dst_p.reshape(_NBANK, _BCH, _NS, _CHUNK)
    zeros = jnp.zeros((_ZROWS, _DH), jnp.float32)

    agg = _sc_agg(x, src_p, dst_p, zeros)

    return pl.pallas_call(
        _mlp_body,
        out_shape=jax.ShapeDtypeStruct((_N, _D), jnp.float32),
    )(x, agg, W1, b1.reshape(1, _D), W2, b2.reshape(1, _D),
      gamma.reshape(1, _D), beta.reshape(1, _D))


# speedup vs baseline: 1.2849x; 1.0042x over previous
import functools

import jax
import jax.numpy as jnp
from jax import lax
from jax.experimental import pallas as pl
from jax.experimental.pallas import tpu as pltpu
from jax.experimental.pallas import tpu_sc as plsc

_N = 10000
_E = 320000
_D = 128
_DH = 64

_NC = 2
_NS = 16
_CHUNK = 128
_BCH = 40
_NBANK = 4
_CPW = _NBANK * _BCH
_EPAD = _NS * _CPW * _CHUNK
_ACC_ROWS = 10256
_ZROWS = 648
_TAIL = _ACC_ROWS - 15 * _ZROWS
_XROWS = 632
_XTAIL = _N - 15 * _XROWS


@functools.partial(
    pl.kernel,
    mesh=plsc.VectorSubcoreMesh(core_axis_name="c", subcore_axis_name="s"),
    compiler_params=pltpu.CompilerParams(use_tc_tiling_on_sc=False),
    out_type=jax.ShapeDtypeStruct((_ACC_ROWS, _D), jnp.float32),
    scratch_types=[
        pltpu.VMEM((2, _BCH, _CHUNK), jnp.int32),
        pltpu.VMEM((2, _BCH, _CHUNK), jnp.int32),
        pltpu.VMEM((3, _CHUNK, _DH), jnp.float32),
        pltpu.VMEM_SHARED((_N, _DH), jnp.float32),
        pltpu.VMEM_SHARED((_ACC_ROWS, _DH), jnp.float32),
        pltpu.SemaphoreType.DMA,
        pltpu.SemaphoreType.DMA,
        pltpu.SemaphoreType.DMA,
        pltpu.SemaphoreType.DMA,
        pltpu.SemaphoreType.DMA,
        pltpu.SemaphoreType.DMA,
        pltpu.SemaphoreType.DMA,
        pltpu.SemaphoreType.DMA,
        pltpu.SemaphoreType.DMA,
        pltpu.SemaphoreType.DMA,
        pltpu.SemaphoreType.DMA,
    ],
)
def _sc_agg(xc_hbm, src_hbm, dst_hbm, zeros_hbm, out_hbm,
            src_v, dst_v, rows_v, x_sh, acc_sh,
            sem0, sem1, sem2, ssem0, ssem1, ssem2, bsem0, bsem1,
            gsem0, gsem1, gsem2):
    sems = (sem0, sem1, sem2)
    gsems = (gsem0, gsem1, gsem2)
    ssems = (ssem0, ssem1, ssem2)
    bsems = (bsem0, bsem1)
    cid = lax.axis_index("c")
    sid = lax.axis_index("s")

    @pl.when(sid < 15)
    def _():
        pltpu.sync_copy(xc_hbm.at[pl.ds(sid * _XROWS, _XROWS),
                                  pl.ds(cid * _DH, _DH)],
                        x_sh.at[pl.ds(sid * _XROWS, _XROWS)])
        pltpu.sync_copy(zeros_hbm, acc_sh.at[pl.ds(sid * _ZROWS, _ZROWS)])

    @pl.when(sid == 15)
    def _():
        pltpu.sync_copy(xc_hbm.at[pl.ds(15 * _XROWS, _XTAIL),
                                  pl.ds(cid * _DH, _DH)],
                        x_sh.at[pl.ds(15 * _XROWS, _XTAIL)])
        pltpu.sync_copy(zeros_hbm.at[pl.ds(0, _TAIL)],
                        acc_sh.at[pl.ds(15 * _ZROWS, _TAIL)])

    for k in range(2):
        pltpu.async_copy(src_hbm.at[k, :, sid], src_v.at[k], bsems[k])
        pltpu.async_copy(dst_hbm.at[k, :, sid], dst_v.at[k], bsems[k])
    plsc.subcore_barrier()

    for k in range(_NBANK):
        s = k % 2
        pltpu.make_async_copy(src_hbm.at[k, :, sid], src_v.at[s],
                              bsems[s]).wait()
        pltpu.make_async_copy(dst_hbm.at[k, :, sid], dst_v.at[s],
                              bsems[s]).wait()

        for b in range(2):
            pltpu.async_copy(x_sh.at[src_v.at[s, b, pl.ds(0, 64)]],
                             rows_v.at[b, pl.ds(0, 64)], sems[b])
            pltpu.async_copy(x_sh.at[src_v.at[s, b, pl.ds(64, 64)]],
                             rows_v.at[b, pl.ds(64, 64)], gsems[b])

        def step(g, carry):
            for b in range(3):
                j = g * 3 + b
                bn = (b + 2) % 3
                pltpu.make_async_copy(
                    x_sh.at[src_v.at[s, j, pl.ds(0, 64)]],
                    rows_v.at[b, pl.ds(0, 64)], sems[b]).wait()
                pltpu.make_async_copy(
                    x_sh.at[src_v.at[s, j, pl.ds(64, 64)]],
                    rows_v.at[b, pl.ds(64, 64)], gsems[b]).wait()
                pltpu.async_copy(rows_v.at[b], acc_sh.at[dst_v.at[s, j]],
                                 ssems[b], add=True)
                if b == 0:
                    @pl.when(g > 0)
                    def _():
                        pltpu.make_async_copy(
                            rows_v.at[bn], acc_sh.at[dst_v.at[s, j - 1]],
                            ssems[bn]).wait()
                else:
                    pltpu.make_async_copy(
                        rows_v.at[bn], acc_sh.at[dst_v.at[s, j - 1]],
                        ssems[bn]).wait()
                if b == 2:
                    @pl.when(g < _BCH // 3 - 1)
                    def _():
                        pltpu.async_copy(x_sh.at[src_v.at[s, j + 2, pl.ds(0, 64)]],
                                         rows_v.at[bn, pl.ds(0, 64)], sems[bn])
                        pltpu.async_copy(
                            x_sh.at[src_v.at[s, j + 2, pl.ds(64, 64)]],
                            rows_v.at[bn, pl.ds(64, 64)], gsems[bn])
                else:
                    pltpu.async_copy(x_sh.at[src_v.at[s, j + 2, pl.ds(0, 64)]],
                                     rows_v.at[bn, pl.ds(0, 64)], sems[bn])
                    pltpu.async_copy(
                        x_sh.at[src_v.at[s, j + 2, pl.ds(64, 64)]],
                        rows_v.at[bn, pl.ds(64, 64)], gsems[bn])
            return carry

        lax.fori_loop(0, _BCH // 3, step, 0)
        j = _BCH - 1
        pltpu.make_async_copy(
            x_sh.at[src_v.at[s, j, pl.ds(0, 64)]],
            rows_v.at[0, pl.ds(0, 64)], sems[0]).wait()
        pltpu.make_async_copy(
            x_sh.at[src_v.at[s, j, pl.ds(64, 64)]],
            rows_v.at[0, pl.ds(64, 64)], gsems[0]).wait()
        pltpu.async_copy(rows_v.at[0], acc_sh.at[dst_v.at[s, j]],
                         ssems[0], add=True)
        pltpu.make_async_copy(
            rows_v.at[2], acc_sh.at[dst_v.at[s, j - 1]], ssems[2]).wait()
        pltpu.make_async_copy(
            rows_v.at[0], acc_sh.at[dst_v.at[s, j]], ssems[0]).wait()
        if k + 2 < _NBANK:
            pltpu.async_copy(src_hbm.at[k + 2, :, sid], src_v.at[s], bsems[s])
            pltpu.async_copy(dst_hbm.at[k + 2, :, sid], dst_v.at[s], bsems[s])

    plsc.subcore_barrier()

    @pl.when(sid < 15)
    def _():
        pltpu.sync_copy(acc_sh.at[pl.ds(sid * _ZROWS, _ZROWS)],
                        out_hbm.at[pl.ds(sid * _ZROWS, _ZROWS),
                                   pl.ds(cid * _DH, _DH)])

    @pl.when(sid == 15)
    def _():
        pltpu.sync_copy(acc_sh.at[pl.ds(15 * _ZROWS, _TAIL)],
                        out_hbm.at[pl.ds(15 * _ZROWS, _TAIL),
                                   pl.ds(cid * _DH, _DH)])


def _mlp_body(x_ref, p_ref, w1_ref, b1_ref, w2_ref, b2_ref, g_ref, be_ref,
              o_ref):
    h = x_ref[...] + p_ref[:_N]
    h = lax.dot_general(h, w1_ref[...], (((1,), (1,)), ((), ())),
                        preferred_element_type=jnp.float32) + b1_ref[...]
    h = jnp.maximum(h, 0.0)
    h = lax.dot_general(h, w2_ref[...], (((1,), (1,)), ((), ())),
                        preferred_element_type=jnp.float32) + b2_ref[...]
    h = jnp.maximum(h, 0.0)
    mean = jnp.mean(h, axis=0, keepdims=True)
    var = jnp.mean(jnp.square(h - mean), axis=0, keepdims=True)
    o_ref[...] = (h - mean) * lax.rsqrt(var + 1e-5) * g_ref[...] + be_ref[...]


def kernel(x, edge_index, W1, b1, W2, b2, gamma, beta):
    src = edge_index[0].astype(jnp.int32)
    dst = edge_index[1].astype(jnp.int32)
    pad = _EPAD - _E
    src_p = jnp.concatenate([src, jnp.zeros((pad,), jnp.int32)])
    dst_fill = _N + (jnp.arange(pad, dtype=jnp.int32) % (_ACC_ROWS - _N))
    dst_p = jnp.concatenate([dst, dst_fill])
    src_p = src_p.reshape(_NBANK, _BCH, _NS, _CHUNK)
    dst_p = dst_p.reshape(_NBANK, _BCH, _NS, _CHUNK)
    zeros = jnp.zeros((_ZROWS, _DH), jnp.float32)

    agg = _sc_agg(x, src_p, dst_p, zeros)

    return pl.pallas_call(
        _mlp_body,
        out_shape=jax.ShapeDtypeStruct((_N, _D), jnp.float32),
    )(x, agg, W1, b1.reshape(1, _D), W2, b2.reshape(1, _D),
      gamma.reshape(1, _D), beta.reshape(1, _D))
